# baseline (speedup 1.0000x reference)
import functools

import jax
import jax.numpy as jnp
from jax import lax
from jax.experimental import pallas as pl
from jax.experimental.pallas import tpu as pltpu
from jax.experimental.pallas import tpu_sc as plsc

_B, _N, _K = 128, 512, 512
_L = 16
_NW = 16
_CHUNK = (_B * _K) // _NW
_VPC = _CHUNK // _L
_GATHER = 128
_NG = _CHUNK // _GATHER
_VR_PER_B = _K // _L
_BATCH_PER_W = _CHUNK // _K


def _sc_body(se_hbm, row_hbm, col_hbm, out_hbm,
             rows_v, cols_v, idx_v, vals_v, shared, red_v, stage_v, sem):
    wid = lax.axis_index("s")
    base = wid * _CHUNK
    pltpu.sync_copy(row_hbm.at[pl.ds(base, _CHUNK)], rows_v)
    pltpu.sync_copy(col_hbm.at[pl.ds(base, _CHUNK)], cols_v)

    for i in range(_VPC):
        b = wid * _BATCH_PER_W + (i // _VR_PER_B)
        r = rows_v[pl.ds(i * _L, _L)]
        c = cols_v[pl.ds(i * _L, _L)]
        flat = r * _N + c + b * (_N * _N)
        j = i // (_GATHER // _L)
        t = i % (_GATHER // _L)
        idx_v[j, pl.ds(t * _L, _L)] = flat

    copies = [pltpu.async_copy(se_hbm.at[idx_v.at[j]], vals_v.at[j], sem)
              for j in range(_NG)]
    for cp in copies:
        cp.wait()

    acc = jnp.zeros((_L,), jnp.float32)
    for j in range(_NG):
        for t in range(_GATHER // _L):
            acc = acc + vals_v[j, pl.ds(t * _L, _L)]

    stage_v[...] = acc
    pltpu.sync_copy(stage_v, out_hbm.at[wid])


def _red_body(part_hbm, out_hbm, red_v, stage_v):
    wid = lax.axis_index("s")

    @pl.when(wid == 0)
    def _():
        pltpu.sync_copy(part_hbm, red_v)
        tot = jnp.zeros((_L,), jnp.float32)
        for w in range(_NW):
            tot = tot + red_v[w, :]
        s = tot[0]
        for i in range(1, _L):
            s = s + tot[i]
        s = s * (1.0 / float(_B * _K))
        stage_v[...] = jnp.broadcast_to(s, (_L,))
        pltpu.sync_copy(stage_v, out_hbm)


_red_call = functools.partial(
    pl.kernel,
    mesh=plsc.VectorSubcoreMesh(core_axis_name="c", subcore_axis_name="s",
                                num_cores=1),
    out_type=jax.ShapeDtypeStruct((_L,), jnp.float32),
    scratch_types=[
        pltpu.VMEM((_NW, _L), jnp.float32),
        pltpu.VMEM((_L,), jnp.float32),
    ],
)(_red_body)


_sc_call = functools.partial(
    pl.kernel,
    mesh=plsc.VectorSubcoreMesh(core_axis_name="c", subcore_axis_name="s",
                                num_cores=1),
    out_type=jax.ShapeDtypeStruct((_NW, _L), jnp.float32),
    scratch_types=[
        pltpu.VMEM((_CHUNK,), jnp.int32),
        pltpu.VMEM((_CHUNK,), jnp.int32),
        pltpu.VMEM((_NG, _GATHER), jnp.int32),
        pltpu.VMEM((_NG, _GATHER), jnp.float32),
        pltpu.VMEM_SHARED((_NW, _L), jnp.float32),
        pltpu.VMEM((_NW, _L), jnp.float32),
        pltpu.VMEM((_L,), jnp.float32),
        pltpu.SemaphoreType.DMA,
    ],
)(_sc_body)


def kernel(squared_error, row_idx, col_idx):
    se_flat = squared_error.reshape(-1)
    rows = row_idx.astype(jnp.int32).reshape(-1)
    cols = col_idx.astype(jnp.int32).reshape(-1)
    partials = _sc_call(se_flat, rows, cols)
    out = _red_call(partials)
    return out[0]


# speedup vs baseline: 3.7805x; 3.7805x over previous
import functools

import jax
import jax.numpy as jnp
from jax import lax
from jax.experimental import pallas as pl
from jax.experimental.pallas import tpu as pltpu
from jax.experimental.pallas import tpu_sc as plsc

_B, _N, _K = 128, 512, 512
_L = 16
_NW = 16
_CHUNK = (_B * _K) // _NW
_VPC = _CHUNK // _L
_GATHER = 128
_NG = _CHUNK // _GATHER
_VR_PER_B = _K // _L
_BATCH_PER_W = _CHUNK // _K


def _sc_body(se_hbm, row_hbm, col_hbm, out_hbm,
             rows_v, cols_v, idx_v, vals_v, shared, red_v, stage_v, sem):
    wid = lax.axis_index("s")
    base = wid * _CHUNK
    pltpu.sync_copy(row_hbm.at[pl.ds(base, _CHUNK)], rows_v)
    pltpu.sync_copy(col_hbm.at[pl.ds(base, _CHUNK)], cols_v)

    for i in range(_VPC):
        b = wid * _BATCH_PER_W + (i // _VR_PER_B)
        r = rows_v[pl.ds(i * _L, _L)]
        c = cols_v[pl.ds(i * _L, _L)]
        flat = ((r >> 3) * (8 * 128 * (_N // 128)) + (c >> 7) * (8 * 128)
                + (r & 7) * 128 + (c & 127) + b * (_N * _N))
        j = i // (_GATHER // _L)
        t = i % (_GATHER // _L)
        idx_v[j, pl.ds(t * _L, _L)] = flat

    copies = [pltpu.async_copy(se_hbm.at[idx_v.at[j]], vals_v.at[j], sem)
              for j in range(_NG)]
    for cp in copies:
        cp.wait()

    acc = jnp.zeros((_L,), jnp.float32)
    for j in range(_NG):
        for t in range(_GATHER // _L):
            acc = acc + vals_v[j, pl.ds(t * _L, _L)]

    stage_v[...] = acc
    pltpu.sync_copy(stage_v, out_hbm.at[wid])


def _red_body(part_hbm, out_hbm, red_v, stage_v):
    wid = lax.axis_index("s")

    @pl.when(wid == 0)
    def _():
        pltpu.sync_copy(part_hbm, red_v)
        tot = jnp.zeros((_L,), jnp.float32)
        for w in range(_NW):
            tot = tot + red_v[w, :]
        s = tot[0]
        for i in range(1, _L):
            s = s + tot[i]
        s = s * (1.0 / float(_B * _K))
        stage_v[...] = jnp.broadcast_to(s, (_L,))
        pltpu.sync_copy(stage_v, out_hbm)


_red_call = functools.partial(
    pl.kernel,
    mesh=plsc.VectorSubcoreMesh(core_axis_name="c", subcore_axis_name="s",
                                num_cores=1),
    out_type=jax.ShapeDtypeStruct((_L,), jnp.float32),
    scratch_types=[
        pltpu.VMEM((_NW, _L), jnp.float32),
        pltpu.VMEM((_L,), jnp.float32),
    ],
)(_red_body)


_sc_call = functools.partial(
    pl.kernel,
    mesh=plsc.VectorSubcoreMesh(core_axis_name="c", subcore_axis_name="s",
                                num_cores=1),
    out_type=jax.ShapeDtypeStruct((_NW, _L), jnp.float32),
    scratch_types=[
        pltpu.VMEM((_CHUNK,), jnp.int32),
        pltpu.VMEM((_CHUNK,), jnp.int32),
        pltpu.VMEM((_NG, _GATHER), jnp.int32),
        pltpu.VMEM((_NG, _GATHER), jnp.float32),
        pltpu.VMEM_SHARED((_NW, _L), jnp.float32),
        pltpu.VMEM((_NW, _L), jnp.float32),
        pltpu.VMEM((_L,), jnp.float32),
        pltpu.SemaphoreType.DMA,
    ],
)(_sc_body)


def kernel(squared_error, row_idx, col_idx):
    se_flat = (squared_error
               .reshape(_B, _N // 8, 8, _N // 128, 128)
               .transpose(0, 1, 3, 2, 4)
               .reshape(-1))
    rows = row_idx.astype(jnp.int32).reshape(-1)
    cols = col_idx.astype(jnp.int32).reshape(-1)
    partials = _sc_call(se_flat, rows, cols)
    out = _red_call(partials)
    return out[0]


# speedup vs baseline: 4.6789x; 1.2376x over previous
import functools

import jax
import jax.numpy as jnp
from jax import lax
from jax.experimental import pallas as pl
from jax.experimental.pallas import tpu as pltpu
from jax.experimental.pallas import tpu_sc as plsc

_B, _N, _K = 128, 512, 512
_L = 16
_NC = 2
_NS = 16
_NW = _NC * _NS
_CHUNK = (_B * _K) // _NW
_VPC = _CHUNK // _L
_GATHER = 128
_NG = _CHUNK // _GATHER
_VR_PER_B = _K // _L
_BATCH_PER_W = _CHUNK // _K


def _sc_body(se_hbm, row_hbm, col_hbm, out_hbm,
             rows_v, cols_v, idx_v, vals_v, stage_v, sem):
    wid = lax.axis_index("s") * _NC + lax.axis_index("c")
    base = wid * _CHUNK
    pltpu.sync_copy(row_hbm.at[pl.ds(base, _CHUNK)], rows_v)
    pltpu.sync_copy(col_hbm.at[pl.ds(base, _CHUNK)], cols_v)

    for i in range(_VPC):
        b = wid * _BATCH_PER_W + (i // _VR_PER_B)
        r = rows_v[pl.ds(i * _L, _L)]
        c = cols_v[pl.ds(i * _L, _L)]
        flat = ((r >> 3) * (8 * 128 * (_N // 128)) + (c >> 7) * (8 * 128)
                + (r & 7) * 128 + (c & 127) + b * (_N * _N))
        j = i // (_GATHER // _L)
        t = i % (_GATHER // _L)
        idx_v[j, pl.ds(t * _L, _L)] = flat

    copies = [pltpu.async_copy(se_hbm.at[idx_v.at[j]], vals_v.at[j], sem)
              for j in range(_NG)]
    for cp in copies:
        cp.wait()

    acc = jnp.zeros((_L,), jnp.float32)
    for j in range(_NG):
        for t in range(_GATHER // _L):
            acc = acc + vals_v[j, pl.ds(t * _L, _L)]

    stage_v[...] = acc
    pltpu.sync_copy(stage_v, out_hbm.at[wid])


_sc_call = functools.partial(
    pl.kernel,
    mesh=plsc.VectorSubcoreMesh(core_axis_name="c", subcore_axis_name="s",
                                num_cores=_NC),
    out_type=jax.ShapeDtypeStruct((_NW, _L), jnp.float32),
    scratch_types=[
        pltpu.VMEM((_CHUNK,), jnp.int32),
        pltpu.VMEM((_CHUNK,), jnp.int32),
        pltpu.VMEM((_NG, _GATHER), jnp.int32),
        pltpu.VMEM((_NG, _GATHER), jnp.float32),
        pltpu.VMEM((_L,), jnp.float32),
        pltpu.SemaphoreType.DMA,
    ],
)(_sc_body)


def _tc_red_body(p_ref, o_ref):
    o_ref[0, 0] = jnp.sum(p_ref[...]) * (1.0 / float(_B * _K))


_tc_red = pl.pallas_call(
    _tc_red_body,
    out_shape=jax.ShapeDtypeStruct((1, 1), jnp.float32),
    out_specs=pl.BlockSpec(memory_space=pltpu.SMEM),
)


def kernel(squared_error, row_idx, col_idx):
    se_flat = (squared_error
               .reshape(_B, _N // 8, 8, _N // 128, 128)
               .transpose(0, 1, 3, 2, 4)
               .reshape(-1))
    rows = row_idx.astype(jnp.int32).reshape(-1)
    cols = col_idx.astype(jnp.int32).reshape(-1)
    partials = _sc_call(se_flat, rows, cols)
    out = _tc_red(partials)
    return out[0, 0]


# speedup vs baseline: 4.7391x; 1.0129x over previous
import functools

import jax
import jax.numpy as jnp
from jax import lax
from jax.experimental import pallas as pl
from jax.experimental.pallas import tpu as pltpu
from jax.experimental.pallas import tpu_sc as plsc

_B, _N, _K = 128, 512, 512
_L = 16
_NC = 2
_NS = 16
_NW = _NC * _NS
_CHUNK = (_B * _K) // _NW
_VPC = _CHUNK // _L
_GATHER = 128
_NG = _CHUNK // _GATHER
_VR_PER_B = _K // _L
_BATCH_PER_W = _CHUNK // _K


def _sc_body(se_hbm, row_hbm, col_hbm, out_hbm,
             rows_v, cols_v, idx_v, vals_v, stage_v, sem_in, sems):
    wid = lax.axis_index("s") * _NC + lax.axis_index("c")
    base = wid * _CHUNK
    cp_r = pltpu.async_copy(row_hbm.at[pl.ds(base, _CHUNK)], rows_v, sem_in)
    cp_c = pltpu.async_copy(col_hbm.at[pl.ds(base, _CHUNK)], cols_v, sem_in)
    cp_r.wait()
    cp_c.wait()

    copies = []
    for j in range(_NG):
        for t in range(_GATHER // _L):
            i = j * (_GATHER // _L) + t
            b = wid * _BATCH_PER_W + (i // _VR_PER_B)
            r = rows_v[pl.ds(i * _L, _L)]
            c = cols_v[pl.ds(i * _L, _L)]
            flat = ((r >> 3) * (8 * 128 * (_N // 128)) + (c >> 7) * (8 * 128)
                    + (r & 7) * 128 + (c & 127) + b * (_N * _N))
            idx_v[j, pl.ds(t * _L, _L)] = flat
        copies.append(
            pltpu.async_copy(se_hbm.at[idx_v.at[j]], vals_v.at[j], sems.at[j]))

    for cp in copies:
        cp.wait()
    acc = jnp.zeros((_L,), jnp.float32)
    for j in range(_NG):
        for t in range(_GATHER // _L):
            acc = acc + vals_v[j, pl.ds(t * _L, _L)]

    stage_v[...] = acc
    pltpu.sync_copy(stage_v, out_hbm.at[wid])


_sc_call = functools.partial(
    pl.kernel,
    mesh=plsc.VectorSubcoreMesh(core_axis_name="c", subcore_axis_name="s",
                                num_cores=_NC),
    out_type=jax.ShapeDtypeStruct((_NW, _L), jnp.float32),
    scratch_types=[
        pltpu.VMEM((_CHUNK,), jnp.int32),
        pltpu.VMEM((_CHUNK,), jnp.int32),
        pltpu.VMEM((_NG, _GATHER), jnp.int32),
        pltpu.VMEM((_NG, _GATHER), jnp.float32),
        pltpu.VMEM((_L,), jnp.float32),
        pltpu.SemaphoreType.DMA,
        pltpu.SemaphoreType.DMA((_NG,)),
    ],
)(_sc_body)


def _tc_red_body(p_ref, o_ref):
    o_ref[0, 0] = jnp.sum(p_ref[...]) * (1.0 / float(_B * _K))


_tc_red = pl.pallas_call(
    _tc_red_body,
    out_shape=jax.ShapeDtypeStruct((1, 1), jnp.float32),
    out_specs=pl.BlockSpec(memory_space=pltpu.SMEM),
)


def kernel(squared_error, row_idx, col_idx):
    se_flat = (squared_error
               .reshape(_B, _N // 8, 8, _N // 128, 128)
               .transpose(0, 1, 3, 2, 4)
               .reshape(-1))
    rows = row_idx.astype(jnp.int32).reshape(-1)
    cols = col_idx.astype(jnp.int32).reshape(-1)
    partials = _sc_call(se_flat, rows, cols)
    out = _tc_red(partials)
    return out[0, 0]


# speedup vs baseline: 4.8027x; 1.0134x over previous
import functools

import jax
import jax.numpy as jnp
from jax import lax
from jax.experimental import pallas as pl
from jax.experimental.pallas import tpu as pltpu
from jax.experimental.pallas import tpu_sc as plsc

_B, _N, _K = 128, 512, 512
_L = 16
_NC = 2
_NS = 16
_NW = _NC * _NS
_CHUNK = (_B * _K) // _NW
_VPC = _CHUNK // _L
_GATHER = 128
_NG = _CHUNK // _GATHER
_VR_PER_B = _K // _L
_BATCH_PER_W = _CHUNK // _K


def _sc_body(se_hbm, row_hbm, col_hbm, out_hbm,
             rows_v, cols_v, idx_v, vals_v, stage_v, sem_in, sems):
    wid = lax.axis_index("s") * _NC + lax.axis_index("c")
    base = wid * _CHUNK
    half = _CHUNK // 2
    cps = [pltpu.async_copy(row_hbm.at[pl.ds(base, half)],
                            rows_v.at[pl.ds(0, half)], sem_in),
           pltpu.async_copy(col_hbm.at[pl.ds(base, half)],
                            cols_v.at[pl.ds(0, half)], sem_in),
           pltpu.async_copy(row_hbm.at[pl.ds(base + half, half)],
                            rows_v.at[pl.ds(half, half)], sem_in),
           pltpu.async_copy(col_hbm.at[pl.ds(base + half, half)],
                            cols_v.at[pl.ds(half, half)], sem_in)]
    cps[0].wait()
    cps[1].wait()

    copies = []
    for j in range(_NG):
        if j == _NG // 2:
            cps[2].wait()
            cps[3].wait()
        for t in range(_GATHER // _L):
            i = j * (_GATHER // _L) + t
            b = wid * _BATCH_PER_W + (i // _VR_PER_B)
            r = rows_v[pl.ds(i * _L, _L)]
            c = cols_v[pl.ds(i * _L, _L)]
            flat = ((r >> 3) * (8 * 128 * (_N // 128)) + (c >> 7) * (8 * 128)
                    + (r & 7) * 128 + (c & 127) + b * (_N * _N))
            idx_v[j, pl.ds(t * _L, _L)] = flat
        copies.append(
            pltpu.async_copy(se_hbm.at[idx_v.at[j]], vals_v.at[j], sems.at[j]))

    for cp in copies:
        cp.wait()
    accs = [jnp.zeros((_L,), jnp.float32) for _ in range(4)]
    n = 0
    for j in range(_NG):
        for t in range(_GATHER // _L):
            accs[n & 3] = accs[n & 3] + vals_v[j, pl.ds(t * _L, _L)]
            n += 1

    stage_v[...] = (accs[0] + accs[1]) + (accs[2] + accs[3])
    pltpu.sync_copy(stage_v, out_hbm.at[wid])


_sc_call = functools.partial(
    pl.kernel,
    mesh=plsc.VectorSubcoreMesh(core_axis_name="c", subcore_axis_name="s",
                                num_cores=_NC),
    out_type=jax.ShapeDtypeStruct((_NW, _L), jnp.float32),
    scratch_types=[
        pltpu.VMEM((_CHUNK,), jnp.int32),
        pltpu.VMEM((_CHUNK,), jnp.int32),
        pltpu.VMEM((_NG, _GATHER), jnp.int32),
        pltpu.VMEM((_NG, _GATHER), jnp.float32),
        pltpu.VMEM((_L,), jnp.float32),
        pltpu.SemaphoreType.DMA,
        pltpu.SemaphoreType.DMA((_NG,)),
    ],
)(_sc_body)


def _tc_red_body(p_ref, o_ref):
    o_ref[0, 0] = jnp.sum(p_ref[...]) * (1.0 / float(_B * _K))


_tc_red = pl.pallas_call(
    _tc_red_body,
    out_shape=jax.ShapeDtypeStruct((1, 1), jnp.float32),
    out_specs=pl.BlockSpec(memory_space=pltpu.SMEM),
)


def kernel(squared_error, row_idx, col_idx):
    se_flat = (squared_error
               .reshape(_B, _N // 8, 8, _N // 128, 128)
               .transpose(0, 1, 3, 2, 4)
               .reshape(-1))
    rows = row_idx.astype(jnp.int32).reshape(-1)
    cols = col_idx.astype(jnp.int32).reshape(-1)
    partials = _sc_call(se_flat, rows, cols)
    out = _tc_red(partials)
    return out[0, 0]
